# baseline (speedup 1.0000x reference)
import functools

import jax
import jax.numpy as jnp
from jax import lax
from jax.experimental import pallas as pl
from jax.experimental.pallas import tpu as pltpu
from jax.experimental.pallas import tpu_sc as plsc

LANES = 16


def _sum_splat(v):
    idx = lax.iota(jnp.int32, 16)
    for k in (8, 4, 2, 1):
        v = v + v.at[jnp.bitwise_xor(idx, jnp.int32(k))].get(
            mode="promise_in_bounds")
    return v


def _rsqrt_newton(x):
    i = lax.bitcast_convert_type(x, jnp.int32)
    i = jnp.int32(0x5F3759DF) - (i >> 1)
    y = lax.bitcast_convert_type(i, jnp.float32)
    half = x * jnp.float32(0.5)
    for _ in range(3):
        y = y * (jnp.float32(1.5) - half * y * y)
    return y


def _make_sc_kernel(B, L, V, D):
    info = plsc.get_sparse_core_info()
    NC, NS = info.num_cores, info.num_subcores
    NW = NC * NS
    assert B % NW == 0 and D == 64 and L == 50
    bpw = B // NW
    rows_per_grp = 4
    grp_idx = rows_per_grp * L
    split = 96
    G = bpw // rows_per_grp
    assert G % 2 == 0
    ngrp = D // LANES
    ipw = bpw * L
    out_rows = bpw // 2

    mesh = plsc.VectorSubcoreMesh(core_axis_name="c", subcore_axis_name="s")

    @functools.partial(
        pl.kernel,
        mesh=mesh,
        out_type=jax.ShapeDtypeStruct((B // 2, 2 * D), jnp.float32),
        compiler_params=pltpu.CompilerParams(use_tc_tiling_on_sc=False),
        scratch_types=[
            pltpu.VMEM((ipw,), jnp.int32),
            pltpu.VMEM((split, D), jnp.float32),
            pltpu.VMEM((grp_idx - split, D), jnp.float32),
            pltpu.VMEM((split, D), jnp.float32),
            pltpu.VMEM((grp_idx - split, D), jnp.float32),
            pltpu.VMEM((out_rows, 2 * D), jnp.float32),
            pltpu.SemaphoreType.DMA,
            pltpu.SemaphoreType.DMA,
        ],
    )
    def sc_kernel(idx_hbm, table_hbm, out_hbm, idx_v, a0, b0, a1, b1, out_v,
                  sem0, sem1):
        wid = lax.axis_index("s") * NC + lax.axis_index("c")
        pltpu.sync_copy(idx_hbm.at[pl.ds(wid * ipw, ipw)], idx_v)

        def fire(t, bufa, bufb, sem):
            base = pl.multiple_of(t * grp_idx, 8)
            pltpu.async_copy(table_hbm.at[idx_v.at[pl.ds(base, split)]],
                             bufa, sem)
            base2 = pl.multiple_of(t * grp_idx + split, 8)
            pltpu.async_copy(
                table_hbm.at[idx_v.at[pl.ds(base2, grp_idx - split)]],
                bufb, sem)

        def drain(bufa, bufb, sem):
            pltpu.make_async_copy(
                table_hbm.at[idx_v.at[pl.ds(0, split)]], bufa, sem).wait()
            pltpu.make_async_copy(
                table_hbm.at[idx_v.at[pl.ds(0, grp_idx - split)]], bufb,
                sem).wait()

        def compute(bufa, bufb, t):
            for r in range(rows_per_grp):
                accs = None
                for j in range(L):
                    jj = r * L + j
                    buf, off = (bufa, jj) if jj < split else (bufb, jj - split)
                    vecs = [buf[off, pl.ds(LANES * c, LANES)]
                            for c in range(ngrp)]
                    accs = (vecs if accs is None
                            else [a + v for a, v in zip(accs, vecs)])
                sq = accs[0] * accs[0]
                for c in range(1, ngrp):
                    sq = sq + accs[c] * accs[c]
                n2 = jnp.maximum(_sum_splat(sq), jnp.float32(1e-24))
                inv = _rsqrt_newton(n2)
                half = (r % 2) * D
                for c in range(ngrp):
                    out_v[2 * t + r // 2,
                          pl.ds(half + LANES * c, LANES)] = accs[c] * inv

        fire(0, a0, b0, sem0)

        def pair_body(h, _):
            t0 = 2 * h
            fire(t0 + 1, a1, b1, sem1)
            drain(a0, b0, sem0)
            compute(a0, b0, t0)
            fire(jnp.minimum(t0 + 2, G - 1), a0, b0, sem0)
            drain(a1, b1, sem1)
            compute(a1, b1, t0 + 1)
            return 0

        lax.fori_loop(0, G // 2, pair_body, 0)
        drain(a0, b0, sem0)
        pltpu.sync_copy(out_v, out_hbm.at[pl.ds(wid * out_rows, out_rows)])

    return sc_kernel


def kernel(text_idxs, text_len, embedding_table):
    del text_len
    B, L = text_idxs.shape
    V, D = embedding_table.shape
    idx_flat = text_idxs.astype(jnp.int32).reshape(-1)
    sc = _make_sc_kernel(B, L, V, D)
    out2 = sc(idx_flat, embedding_table)
    return out2.reshape(B, D)
